# Initial kernel scaffold; baseline (speedup 1.0000x reference)
import jax
import jax.numpy as jnp
from jax.experimental import pallas as pl


def kernel(x, emb_table, W, b):
    raise NotImplementedError("write your pallas kernel here")


# speedup vs baseline: 2.2035x; 2.2035x over previous
import functools

import jax
import jax.numpy as jnp
from jax import lax
from jax.experimental import pallas as pl
from jax.experimental.pallas import tpu as pltpu
from jax.experimental.pallas import tpu_sc as plsc

NC = 2
NS = 16
NW = NC * NS
LANES = 16

CHUNK = 40


def _sc_pooled_sums(x, emb_table, B, L, D):
  CB = B // NW
  SPB = L // CHUNK
  x2 = x.reshape(B * SPB, CHUNK)

  mesh = plsc.VectorSubcoreMesh(
      core_axis_name="c", subcore_axis_name="s", num_cores=NC,
      num_subcores=NS)

  @functools.partial(
      pl.kernel,
      out_type=jax.ShapeDtypeStruct((B, D), jnp.float32),
      mesh=mesh,
      compiler_params=pltpu.CompilerParams(use_tc_tiling_on_sc=False),
      scratch_types=[
          pltpu.VMEM((CB * SPB, CHUNK), jnp.int32),
          pltpu.VMEM((L, D), jnp.float32),
          pltpu.VMEM((L, D), jnp.float32),
          pltpu.VMEM((CB, D), jnp.float32),
          pltpu.SemaphoreType.DMA,
          pltpu.SemaphoreType.DMA,
      ],
  )
  def sc_kernel(x_hbm, tab_hbm, out_hbm, idx_v, rbuf_a, rbuf_b, sum_v,
                sem_a, sem_b):
    c = lax.axis_index("c")
    s = lax.axis_index("s")
    wid = s * NC + c

    pltpu.sync_copy(x_hbm.at[pl.ds(wid * CB * SPB, CB * SPB)], idx_v)

    def fire(i, rbuf, sem):
      for j in range(SPB):
        pltpu.async_copy(
            tab_hbm.at[idx_v.at[i * SPB + j]],
            rbuf.at[pl.ds(j * CHUNK, CHUNK)],
            sem)

    def wait(rbuf, sem):
      pltpu.make_async_copy(out_hbm.at[pl.ds(0, L)], rbuf, sem).wait()

    def accumulate(i, rbuf):
      accs = [jnp.zeros((LANES,), jnp.float32) for _ in range(8)]
      for k in range(L):
        p = k % 4
        accs[2 * p] = accs[2 * p] + rbuf[k, pl.ds(0, LANES)]
        accs[2 * p + 1] = accs[2 * p + 1] + rbuf[k, pl.ds(LANES, LANES)]
      lo = (accs[0] + accs[2]) + (accs[4] + accs[6])
      hi = (accs[1] + accs[3]) + (accs[5] + accs[7])
      sum_v[i, pl.ds(0, LANES)] = lo
      sum_v[i, pl.ds(LANES, LANES)] = hi

    fire(0, rbuf_a, sem_a)

    @pl.loop(0, CB, step=2)
    def _(i):
      fire(i + 1, rbuf_b, sem_b)
      wait(rbuf_a, sem_a)
      accumulate(i, rbuf_a)

      @pl.when(i + 2 < CB)
      def _():
        fire(i + 2, rbuf_a, sem_a)

      wait(rbuf_b, sem_b)
      accumulate(i + 1, rbuf_b)

    pltpu.sync_copy(sum_v, out_hbm.at[pl.ds(wid * CB, CB)])

  return sc_kernel(x2, emb_table)


def _tc_linear(sums, W, b2, L):
  B, D = sums.shape
  NCLS = W.shape[1]

  def body(s_ref, w_ref, b_ref, o_ref):
    m = s_ref[...] * jnp.float32(1.0 / L)
    o_ref[...] = (
        jnp.dot(m, w_ref[...], preferred_element_type=jnp.float32)
        + b_ref[...])

  return pl.pallas_call(
      body,
      out_shape=jax.ShapeDtypeStruct((B, NCLS), jnp.float32),
  )(sums, W, b2)


def kernel(x, emb_table, W, b):
  B, L = x.shape
  D = emb_table.shape[1]
  x = x.astype(jnp.int32)
  sums = _sc_pooled_sums(x, emb_table, B, L, D)
  return _tc_linear(sums, W, b.reshape(1, -1).astype(jnp.float32), L)


# speedup vs baseline: 2.2039x; 1.0002x over previous
import functools

import jax
import jax.numpy as jnp
from jax import lax
from jax.experimental import pallas as pl
from jax.experimental.pallas import tpu as pltpu
from jax.experimental.pallas import tpu_sc as plsc

NC = 2
NS = 16
NW = NC * NS
LANES = 16

CHUNK = 40


def _sc_pooled_sums(x, emb_table, B, L, D):
  CB = B // NW
  SPB = L // CHUNK
  x1 = x.reshape(-1)

  mesh = plsc.VectorSubcoreMesh(
      core_axis_name="c", subcore_axis_name="s", num_cores=NC,
      num_subcores=NS)

  @functools.partial(
      pl.kernel,
      out_type=jax.ShapeDtypeStruct((B, D), jnp.float32),
      mesh=mesh,
      compiler_params=pltpu.CompilerParams(use_tc_tiling_on_sc=False),
      scratch_types=[
          pltpu.VMEM((CB * L,), jnp.int32),
          pltpu.VMEM((L, D), jnp.float32),
          pltpu.VMEM((L, D), jnp.float32),
          pltpu.VMEM((CB, D), jnp.float32),
          pltpu.SemaphoreType.DMA,
          pltpu.SemaphoreType.DMA,
      ],
  )
  def sc_kernel(x_hbm, tab_hbm, out_hbm, idx_v, rbuf_a, rbuf_b, sum_v,
                sem_a, sem_b):
    c = lax.axis_index("c")
    s = lax.axis_index("s")
    wid = s * NC + c

    pltpu.sync_copy(x_hbm.at[pl.ds(wid * CB * L, CB * L)], idx_v)

    def fire(i, rbuf, sem):
      for j in range(SPB):
        pltpu.async_copy(
            tab_hbm.at[idx_v.at[pl.ds(i * L + j * CHUNK, CHUNK)]],
            rbuf.at[pl.ds(j * CHUNK, CHUNK)],
            sem)

    def wait(rbuf, sem):
      pltpu.make_async_copy(out_hbm.at[pl.ds(0, L)], rbuf, sem).wait()

    def accumulate(i, rbuf):
      accs = [jnp.zeros((LANES,), jnp.float32) for _ in range(8)]
      for k in range(L):
        p = k % 4
        accs[2 * p] = accs[2 * p] + rbuf[k, pl.ds(0, LANES)]
        accs[2 * p + 1] = accs[2 * p + 1] + rbuf[k, pl.ds(LANES, LANES)]
      lo = (accs[0] + accs[2]) + (accs[4] + accs[6])
      hi = (accs[1] + accs[3]) + (accs[5] + accs[7])
      sum_v[i, pl.ds(0, LANES)] = lo
      sum_v[i, pl.ds(LANES, LANES)] = hi

    fire(0, rbuf_a, sem_a)

    @pl.loop(0, CB, step=2)
    def _(i):
      fire(i + 1, rbuf_b, sem_b)
      wait(rbuf_a, sem_a)
      accumulate(i, rbuf_a)

      @pl.when(i + 2 < CB)
      def _():
        fire(i + 2, rbuf_a, sem_a)

      wait(rbuf_b, sem_b)
      accumulate(i + 1, rbuf_b)

    pltpu.sync_copy(sum_v, out_hbm.at[pl.ds(wid * CB, CB)])

  return sc_kernel(x1, emb_table)


def _tc_linear(sums, W, b2, L):
  B, D = sums.shape
  NCLS = W.shape[1]

  def body(s_ref, w_ref, b_ref, o_ref):
    m = s_ref[...] * jnp.float32(1.0 / L)
    o_ref[...] = (
        jnp.dot(m, w_ref[...], preferred_element_type=jnp.float32)
        + b_ref[...])

  return pl.pallas_call(
      body,
      out_shape=jax.ShapeDtypeStruct((B, NCLS), jnp.float32),
  )(sums, W, b2)


def kernel(x, emb_table, W, b):
  B, L = x.shape
  D = emb_table.shape[1]
  x = x.astype(jnp.int32)
  sums = _sc_pooled_sums(x, emb_table, B, L, D)
  return _tc_linear(sums, W, b.reshape(1, -1).astype(jnp.float32), L)
